# baseline (speedup 1.0000x reference)
import functools

import jax
import jax.numpy as jnp
from jax import lax
from jax.experimental import pallas as pl
from jax.experimental.pallas import tpu as pltpu
from jax.experimental.pallas import tpu_sc as plsc

VOCAB = 1000
VPAD = 1024
N_EMBD = 32
T = 8
ROWS = 4096 * 8
NW = 32
ROWS_PER_W = ROWS // NW
CHUNK = 128
NCHUNK = ROWS_PER_W // CHUNK
BLK = 4096
NCHUNKS_TC = 4
CROWS = ROWS // NCHUNKS_TC
GRID = CROWS // BLK


def _sc_gather_kernel(table_hbm, idx_hbm, out_hbm, idx_v, rows_v, sem):
    wid = lax.axis_index("s") * 2 + lax.axis_index("c")
    base = wid * NCHUNK
    pltpu.sync_copy(idx_hbm.at[pl.ds(base, NCHUNK)], idx_v)
    for j in range(NCHUNK):
        pltpu.async_copy(table_hbm.at[idx_v.at[j]], rows_v.at[j], sem).wait()
        pltpu.sync_copy(
            rows_v.at[j],
            out_hbm.at[pl.ds(wid * ROWS_PER_W + j * CHUNK, CHUNK)],
        )


@jax.jit
def _sc_gather(tok_table, idx2):
    mesh = plsc.VectorSubcoreMesh(core_axis_name="c", subcore_axis_name="s")
    return pl.kernel(
        _sc_gather_kernel,
        mesh=mesh,
        out_type=jax.ShapeDtypeStruct((ROWS, N_EMBD), jnp.float32),
        scratch_types=[
            pltpu.VMEM((NCHUNK, CHUNK), jnp.int32),
            pltpu.VMEM((NCHUNK, CHUNK, N_EMBD), jnp.float32),
            pltpu.SemaphoreType.DMA,
        ],
        compiler_params=pltpu.CompilerParams(use_tc_tiling_on_sc=False),
    )(tok_table, idx2)


def _tc_head_kernel(x_ref, pos_ref, w_ref, b_ref, t_ref, logits_ref, loss_ref):
    i = pl.program_id(0)
    x = x_ref[...]
    xp = x.reshape(BLK // T, T, N_EMBD) + pos_ref[...][None, :, :]
    xp = xp.reshape(BLK, N_EMBD)
    logits = (
        jnp.dot(xp, w_ref[...], preferred_element_type=jnp.float32,
                precision=lax.Precision.DEFAULT)
        + b_ref[...]
    )
    logits_ref[...] = logits

    viota = lax.broadcasted_iota(jnp.int32, (BLK, VPAD), 1)
    valid = viota < VOCAB
    neg = jnp.where(valid, logits, -jnp.inf)
    rowmax = jnp.max(neg, axis=1, keepdims=True)
    se = jnp.sum(jnp.where(valid, jnp.exp(logits - rowmax), 0.0), axis=1)
    tmask = viota == t_ref[...]
    tlogit = jnp.sum(jnp.where(tmask, logits, 0.0), axis=1)
    bs = jnp.sum(jnp.log(se) + rowmax[:, 0] - tlogit).reshape(1, 1)

    @pl.when(i == 0)
    def _init():
        loss_ref[...] = jnp.zeros((1, 1), jnp.float32)

    loss_ref[...] += bs


@jax.jit
def _tc_head(x, pos_table, Wp, bp, t2):
    return pl.pallas_call(
        _tc_head_kernel,
        grid=(GRID,),
        in_specs=[
            pl.BlockSpec((BLK, N_EMBD), lambda i: (i, 0)),
            pl.BlockSpec((T, N_EMBD), lambda i: (0, 0)),
            pl.BlockSpec((N_EMBD, VPAD), lambda i: (0, 0)),
            pl.BlockSpec((1, VPAD), lambda i: (0, 0)),
            pl.BlockSpec((BLK, 1), lambda i: (i, 0)),
        ],
        out_specs=[
            pl.BlockSpec((BLK, VPAD), lambda i: (i, 0)),
            pl.BlockSpec((1, 1), lambda i: (0, 0)),
        ],
        out_shape=[
            jax.ShapeDtypeStruct((CROWS, VPAD), jnp.float32),
            jax.ShapeDtypeStruct((1, 1), jnp.float32),
        ],
    )(x, pos_table, Wp, bp, t2)


def kernel(idx, targets, tok_table, pos_table, W, b):
    idx2 = idx.reshape(NW * NCHUNK, CHUNK).astype(jnp.int32)
    x = _sc_gather(tok_table, idx2)
    t2 = targets.reshape(ROWS, 1).astype(jnp.int32)
    Wp = jnp.pad(W, ((0, 0), (0, VPAD - VOCAB)))
    bp = jnp.pad(b, (0, VPAD - VOCAB)).reshape(1, VPAD)
    pieces, part = [], None
    for c in range(NCHUNKS_TC):
        sl = slice(c * CROWS, (c + 1) * CROWS)
        padded, s = _tc_head(x[sl], pos_table, Wp, bp, t2[sl])
        pieces.append(padded[:, :VOCAB])
        part = s if part is None else part + s
    logits2 = jnp.concatenate(pieces, axis=0)
    return (logits2, part[0, 0] / ROWS)
